# baseline (speedup 1.0000x reference)
import functools

import jax
import jax.numpy as jnp
from jax import lax
from jax.experimental import pallas as pl
from jax.experimental.pallas import tpu as pltpu
from jax.experimental.pallas import tpu_sc as plsc

F32 = jnp.float32
BF16 = jnp.bfloat16
I32 = jnp.int32
NW = 32
KP1 = 17


def _ceil16(x):
    return (x + 15) // 16 * 16


def _ceil32(x):
    return (x + 31) // 32 * 32


def _mm_body(x_ref, w_ref, o_ref):
    o_ref[...] = lax.dot(x_ref[...], w_ref[...],
                         precision=lax.Precision.HIGHEST,
                         preferred_element_type=F32).astype(o_ref.dtype)


def _tc_matmul(x, w, out_dtype=F32, bm=512):
    m, p = x.shape
    n = w.shape[1]
    return pl.pallas_call(
        _mm_body,
        grid=(m // bm,),
        in_specs=[pl.BlockSpec((bm, p), lambda i: (i, 0)),
                  pl.BlockSpec((p, n), lambda i: (0, 0))],
        out_specs=pl.BlockSpec((bm, n), lambda i: (i, 0)),
        out_shape=jax.ShapeDtypeStruct((m, n), out_dtype),
    )(x, w)


KA = 9
KB = KP1 - KA


def _sc_gather_sum(ya, yb, idx_a, idx_b, bias, b, e, dpad):
    nn = b * e
    bpc = b // 2
    npt = e // 16
    ngroups = dpad // 32
    mesh = plsc.VectorSubcoreMesh(core_axis_name="c", subcore_axis_name="s")

    @functools.partial(
        pl.kernel, mesh=mesh,
        out_type=jax.ShapeDtypeStruct((nn, dpad), F32),
        compiler_params=pltpu.CompilerParams(use_tc_tiling_on_sc=False,
                                             needs_layout_passes=False),
        scratch_types=[
            pltpu.VMEM_SHARED((e * KA, dpad), BF16),
            pltpu.VMEM((npt * KA,), I32),
            pltpu.VMEM((npt * KA, dpad), BF16),
            pltpu.VMEM((npt, dpad), BF16),
            pltpu.VMEM((npt, dpad), F32),
            pltpu.VMEM((dpad,), F32),
            pltpu.SemaphoreType.DMA,
        ],
    )
    def k(ya_hbm, yb_hbm, ia_hbm, ib_hbm, bias_hbm, out_hbm, sp_table,
          idx_v, rows_v, acca_v, out_v, bias_v, sem):
        cid = lax.axis_index("c")
        sid = lax.axis_index("s")
        pltpu.sync_copy(bias_hbm, bias_v)

        def stage(src_hbm, gb, nslots):
            rows = e * nslots
            stg = rows // 16
            pltpu.sync_copy(src_hbm.at[pl.ds(gb * rows + sid * stg, stg)],
                            sp_table.at[pl.ds(sid * stg, stg)])

        def gather(i_hbm, node0, nslots):
            nrows = npt * nslots
            pltpu.sync_copy(i_hbm.at[pl.ds(node0 * nslots, nrows)],
                            idx_v.at[pl.ds(0, nrows)])
            descs = []
            for q in range(nrows // 128):
                descs.append(pltpu.async_copy(
                    sp_table.at[idx_v.at[pl.ds(q * 128, 128)]],
                    rows_v.at[pl.ds(q * 128, 128)], sem))
            for d in descs:
                d.wait()

        def batch_body(j, carry):
            gb = cid * bpc + j
            node0 = gb * e + sid * npt
            stage(ya_hbm, gb, KA)
            plsc.subcore_barrier()
            gather(ia_hbm, node0, KA)
            plsc.subcore_barrier()

            def acc_a(i, carry2):
                r0 = i * KA
                for grp in range(ngroups):
                    a32 = rows_v[r0, pl.ds(32 * grp, 32)]
                    for kk in range(1, KA):
                        a32 = a32 + rows_v[r0 + kk, pl.ds(32 * grp, 32)]
                    acca_v[i, pl.ds(32 * grp, 32)] = a32
                return carry2

            lax.fori_loop(0, npt, acc_a, 0)
            stage(yb_hbm, gb, KB)
            plsc.subcore_barrier()
            gather(ib_hbm, node0, KB)
            plsc.subcore_barrier()

            def acc_b(i, carry2):
                r0 = i * KB
                for grp in range(ngroups):
                    a32 = acca_v[i, pl.ds(32 * grp, 32)]
                    for kk in range(KB):
                        a32 = a32 + rows_v[r0 + kk, pl.ds(32 * grp, 32)]
                    aa, ab = plsc.unpack(a32,
                                         format=plsc.PackFormat.INTERLEAVED)
                    sla = pl.ds(32 * grp, 16)
                    slb = pl.ds(32 * grp + 16, 16)
                    out_v[i, sla] = jnp.maximum(aa + bias_v[sla], 0.0)
                    out_v[i, slb] = jnp.maximum(ab + bias_v[slb], 0.0)
                return carry2

            lax.fori_loop(0, npt, acc_b, 0)
            pltpu.sync_copy(out_v, out_hbm.at[pl.ds(node0, npt)])
            return carry

        lax.fori_loop(0, bpc, batch_body, 0)

    return k(ya, yb, idx_a, idx_b, bias)


def _sc_masked_mean(z, n_arr, b, e, dpad, fdim):
    fpad = _ceil16(fdim)
    nsl = fpad // 16
    rows_chunk = 512
    nch = e // rows_chunk
    mesh = plsc.VectorSubcoreMesh(core_axis_name="c", subcore_axis_name="s")

    @functools.partial(
        pl.kernel, mesh=mesh,
        out_type=jax.ShapeDtypeStruct((b, fpad), F32),
        compiler_params=pltpu.CompilerParams(use_tc_tiling_on_sc=False),
        scratch_types=[
            pltpu.VMEM((rows_chunk, dpad), F32),
            pltpu.VMEM((16,), I32),
            pltpu.VMEM((fpad,), F32),
        ],
    )
    def k(z_hbm, n_hbm, out_hbm, zrows_v, n_v, out_v):
        wid = lax.axis_index("s") * 2 + lax.axis_index("c")

        @pl.when(wid < b)
        def _():
            _masked_mean_worker(z_hbm, n_hbm, out_hbm, zrows_v, n_v, out_v,
                                wid, e, dpad, fdim, nsl, rows_chunk, nch)

    return k(z, n_arr)


def _masked_mean_worker(z_hbm, n_hbm, out_hbm, zrows_v, n_v, out_v, wid, e,
                        dpad, fdim, nsl, rows_chunk, nch):
        pltpu.sync_copy(n_hbm.at[wid], n_v)
        nsplat = n_v[pl.ds(0, 16)]
        iota = lax.iota(I32, 16)
        accs = [jnp.zeros((16,), F32) for _ in range(nsl)]
        for ch in range(nch):
            pltpu.sync_copy(z_hbm.at[pl.ds(wid * e + ch * rows_chunk,
                                           rows_chunk)], zrows_v)

            def ebody(i, carry):
                pred = (ch * rows_chunk + i) < nsplat
                out = []
                for s in range(nsl):
                    lanes_valid = 16 * s + iota < fdim
                    v = jnp.where(pred & lanes_valid,
                                  zrows_v[i, pl.ds(16 * s, 16)], 0.0)
                    out.append(carry[s] + v)
                return tuple(out)

            accs = lax.fori_loop(0, rows_chunk, ebody, tuple(accs))
        inv = 1.0 / jnp.maximum(nsplat, 1).astype(F32)
        for s in range(nsl):
            out_v[pl.ds(16 * s, 16)] = accs[s] * inv
        pltpu.sync_copy(out_v, out_hbm.at[wid])


def _head_body(x_ref, w1_ref, b1_ref, w2_ref, b2_ref, w3_ref, b3_ref,
               lg_ref, pred_ref):
    x = x_ref[...]
    h = jnp.maximum(lax.dot(x, w1_ref[...], precision=lax.Precision.HIGHEST,
                            preferred_element_type=F32) + b1_ref[...], 0.0)
    h = jnp.maximum(lax.dot(h, w2_ref[...], precision=lax.Precision.HIGHEST,
                            preferred_element_type=F32) + b2_ref[...], 0.0)
    lg = lax.dot(h, w3_ref[...], precision=lax.Precision.HIGHEST,
                 preferred_element_type=F32) + b3_ref[...]
    lg_ref[...] = lg
    ncls = lg.shape[1]
    col = lax.broadcasted_iota(I32, lg.shape, 1)
    mx = jnp.max(lg, axis=1, keepdims=True)
    pred_ref[...] = jnp.min(jnp.where(lg >= mx, col, ncls), axis=1,
                            keepdims=True)


def _tc_head(flat, w1, b1, w2, b2, w3, b3):
    b = flat.shape[0]
    ncls = w3.shape[1]
    return pl.pallas_call(
        _head_body,
        out_shape=(jax.ShapeDtypeStruct((b, ncls), F32),
                   jax.ShapeDtypeStruct((b, 1), I32)),
    )(flat, w1, b1, w2, b2, w3, b3)


def _build_wbig(wa, ws, fa, fs, p, out, dpad):
    kk = wa.shape[0] // (fa + fs)
    wa_r = wa.reshape(kk, fa + fs, out)
    ws_r = ws.reshape(kk, fa + fs, out)
    blocks = jnp.concatenate([wa_r, ws_r], axis=2)
    corr = -jnp.concatenate([wa_r[:, fa:, :].sum(0),
                             ws_r[:, fa:, :].sum(0)], axis=1)
    corr_full = jnp.zeros((fa + fs, 2 * out), F32).at[fa:].set(corr)
    wb = jnp.concatenate([blocks, corr_full[None]], axis=0)
    wb = jnp.pad(wb, ((0, 0), (0, p - (fa + fs)), (0, dpad - 2 * out)))
    perm = []
    for s in range(dpad // 32):
        for t in range(16):
            perm.extend((32 * s + t, 32 * s + 16 + t))
    wb = wb[:, :, jnp.array(perm, dtype=I32)]
    wb = wb.transpose(1, 0, 2).reshape(p, KP1 * dpad)
    return wb[:, :KA * dpad], wb[:, KA * dpad:]


def kernel(space_features, all_features, neighbors_matrix, num_entries,
           params):
    b, e, fs0 = space_features.shape
    fa0 = all_features.shape[2]
    kk = neighbors_matrix.shape[2]
    nn = b * e
    nlayers = 6
    layer_out = [params['W%da' % l].shape[1] for l in range(nlayers)]

    nbr = neighbors_matrix.astype(I32)
    idx_a = (nbr[:, :, :KA] * KA
             + jnp.arange(KA, dtype=I32)[None, None, :]).reshape(-1)
    self_row = jnp.broadcast_to(
        (jnp.arange(e, dtype=I32) * KB + (KB - 1))[None, :, None], (b, e, 1))
    idx_b = jnp.concatenate(
        [nbr[:, :, KA:] * KB + jnp.arange(KB - 1, dtype=I32)[None, None, :],
         self_row], axis=2).reshape(-1)

    z = jnp.concatenate([all_features.reshape(nn, fa0),
                         space_features.reshape(nn, fs0)], axis=1)
    fa, fs = fa0, fs0
    for l in range(nlayers):
        out = layer_out[l]
        dpad = _ceil32(2 * out)
        p = z.shape[1]
        wbig_a, wbig_b = _build_wbig(params['W%da' % l], params['W%ds' % l],
                                     fa, fs, p, out, dpad)
        bias = jnp.pad(jnp.concatenate([params['b%da' % l],
                                        params['b%ds' % l]]),
                       (0, dpad - 2 * out))
        ya = _tc_matmul(z, wbig_a, BF16)
        yb = _tc_matmul(z, wbig_b, BF16)
        z = _sc_gather_sum(ya.reshape(nn * KA, dpad),
                           yb.reshape(nn * KB, dpad),
                           idx_a, idx_b, bias, b, e, dpad)
        fa = fs = out

    n_rep = jnp.tile(num_entries.reshape(b, 1).astype(I32), (1, 16))
    flat = _sc_masked_mean(z, n_rep, b, e, z.shape[1], layer_out[-1])
    f1 = jnp.pad(params['fc1_w'], ((0, flat.shape[1] - layer_out[-1]),
                                   (0, 0)))
    logits, pred = _tc_head(flat, f1, params['fc1_b'][None],
                            params['fc2_w'], params['fc2_b'][None],
                            params['fc3_w'], params['fc3_b'][None])
    return logits, pred[:, 0]
